# Initial kernel scaffold; baseline (speedup 1.0000x reference)
import jax
import jax.numpy as jnp
from jax.experimental import pallas as pl


def kernel(gene_data, spatial_edge_index, mask, W1, b1, W2, b2, Wd, bd):
    raise NotImplementedError("write your pallas kernel here")


# speedup vs baseline: 9.9295x; 9.9295x over previous
import functools

import numpy as np
import jax
import jax.numpy as jnp
from jax import lax
from jax.experimental import pallas as pl
from jax.experimental.pallas import tpu as pltpu
from jax.experimental.pallas import tpu_sc as plsc

_N = 10000
_E = 320000
_D0 = 128
_D1 = 128
_D2 = 64
_NP = 10240
_EP = 323584
_CH = 128
_EPT = _EP // 16
_RPT = _NP // 16

_PERM_CACHE = {}


def _tf2x32(k1, k2, x1, x2):
    def rotl(x, d):
        return ((x << np.uint32(d)) | (x >> np.uint32(32 - d))).astype(np.uint32)

    ks = (np.uint32(k1), np.uint32(k2),
          np.uint32(k1) ^ np.uint32(k2) ^ np.uint32(0x1BD11BDA))
    x = [(x1 + ks[0]).astype(np.uint32), (x2 + ks[1]).astype(np.uint32)]
    rots = ((13, 15, 26, 6), (17, 29, 16, 24)) * 3
    for i in range(5):
        for r in rots[i]:
            x[0] = (x[0] + x[1]).astype(np.uint32)
            x[1] = rotl(x[1], r) ^ x[0]
        x[0] = (x[0] + ks[(i + 1) % 3]).astype(np.uint32)
        x[1] = (x[1] + ks[(i + 2) % 3] + np.uint32(i + 1)).astype(np.uint32)
    return x[0], x[1]


def _perm_np():
    if "p" not in _PERM_CACHE:
        key = np.array([0, 1], np.uint32)
        x = np.arange(_N, dtype=np.int32)
        num_rounds = int(np.ceil(3 * np.log(_N) / np.log(2**32 - 1)))
        for _ in range(num_rounds):
            b1, b2 = _tf2x32(key[0], key[1], np.zeros(2, np.uint32),
                             np.arange(2, dtype=np.uint32))
            keys = np.stack([b1, b2], axis=1)
            key, subkey = keys[0], keys[1]
            s1, s2 = _tf2x32(subkey[0], subkey[1], np.zeros(_N, np.uint32),
                             np.arange(_N, dtype=np.uint32))
            x = x[np.argsort(s1 ^ s2, kind="stable")]
        _PERM_CACHE["p"] = x
    return _PERM_CACHE["p"]


_EPW = _EP // 32


def _sc_pass0_body(dst_hbm, perm_hbm, gene_hbm, zeros_hbm, ones_hbm,
                   hist_hbm, genep_hbm,
                   idx_v, ones_v, rows_v, sem, hist_sh):
    c = lax.axis_index("c")
    s = lax.axis_index("s")

    pltpu.sync_copy(zeros_hbm.at[pl.ds(s * _RPT, _RPT)],
                    hist_sh.at[pl.ds(s * _RPT, _RPT)])
    pltpu.sync_copy(ones_hbm, ones_v)

    @pl.when(c == 1)
    def _():
        def body(t, carry):
            base = s * _RPT + t * _CH
            pltpu.sync_copy(perm_hbm.at[pl.ds(base, _CH)], idx_v)
            pltpu.async_copy(gene_hbm.at[idx_v], rows_v, sem).wait()
            pltpu.sync_copy(rows_v, genep_hbm.at[pl.ds(base, _CH)])
            return carry

        lax.fori_loop(0, _RPT // _CH, body, 0)

    plsc.subcore_barrier()
    w = c * 16 + s

    def body(t, carry):
        base = w * _EPW + t * _CH
        pltpu.sync_copy(dst_hbm.at[pl.ds(base, _CH)], idx_v)
        pltpu.sync_copy(ones_v, hist_sh.at[idx_v], add=True)
        return carry

    lax.fori_loop(0, _EPW // _CH, body, 0)
    plsc.subcore_barrier()
    pltpu.sync_copy(hist_sh.at[pl.ds(s * _RPT, _RPT)],
                    hist_hbm.at[c, pl.ds(s * _RPT, _RPT)])


def _sc_pass0(dst_pad, perm_pad, gene_pad, zeros_tab, ones_tab):
    mesh = plsc.VectorSubcoreMesh(core_axis_name="c", subcore_axis_name="s")
    f = pl.kernel(
        _sc_pass0_body,
        out_type=[
            jax.ShapeDtypeStruct((2, _NP, _D0), jnp.float32),
            jax.ShapeDtypeStruct((_NP, _D0), jnp.float32),
        ],
        mesh=mesh,
        scratch_types=[
            pltpu.VMEM((_CH,), jnp.int32),
            pltpu.VMEM((_CH, _D0), jnp.float32),
            pltpu.VMEM((_CH, _D0), jnp.float32),
            pltpu.SemaphoreType.DMA,
            pltpu.VMEM_SHARED((_NP, _D0), jnp.float32),
        ],
    )
    return f(dst_pad, perm_pad, gene_pad, zeros_tab, ones_tab)


def _sc_agg_body(d, src_hbm, dst_hbm, table_hbm, out_hbm,
                 sidx_v, aidx_v, didx_v, rows_v, sem, acc_sh):
    c = lax.axis_index("c")
    s = lax.axis_index("s")
    pltpu.sync_copy(table_hbm.at[pl.ds(c * _NP + s * _RPT, _RPT)],
                    acc_sh.at[pl.ds(s * _RPT, _RPT)])
    plsc.subcore_barrier()
    off = c * _NP

    def body(t, carry):
        base = s * _EPT + t * _CH
        pltpu.sync_copy(src_hbm.at[pl.ds(base, _CH)], sidx_v)
        pltpu.sync_copy(dst_hbm.at[pl.ds(base, _CH)], didx_v)
        for k in range(_CH // 16):
            aidx_v[pl.ds(k * 16, 16)] = sidx_v[pl.ds(k * 16, 16)] + off
        pltpu.async_copy(table_hbm.at[aidx_v], rows_v, sem).wait()
        pltpu.sync_copy(rows_v, acc_sh.at[didx_v], add=True)
        return carry

    lax.fori_loop(0, _EPT // _CH, body, 0)
    plsc.subcore_barrier()
    pltpu.sync_copy(acc_sh.at[pl.ds(s * _RPT, _RPT)],
                    out_hbm.at[c, pl.ds(s * _RPT, _RPT)])


def _sc_agg(src_pad, dst_pad, table, d):
    mesh = plsc.VectorSubcoreMesh(core_axis_name="c", subcore_axis_name="s")
    f = pl.kernel(
        functools.partial(_sc_agg_body, d),
        out_type=jax.ShapeDtypeStruct((2, _NP, d), jnp.float32),
        mesh=mesh,
        scratch_types=[
            pltpu.VMEM((_CH,), jnp.int32),
            pltpu.VMEM((_CH,), jnp.int32),
            pltpu.VMEM((_CH,), jnp.int32),
            pltpu.VMEM((_CH, d), jnp.float32),
            pltpu.SemaphoreType.DMA,
            pltpu.VMEM_SHARED((_NP, d), jnp.float32),
        ],
    )
    return f(src_pad, dst_pad, table)


_BA = 640


def _tca_body(gene_ref, genep_ref, hist_ref, w1_ref, tab_ref, dis_ref):
    dis = lax.rsqrt(1.0 + hist_ref[0][:, 0:1] + hist_ref[1][:, 0:1])
    w1 = w1_ref[...]
    tab_ref[0] = dis * jnp.dot(gene_ref[...], w1,
                               preferred_element_type=jnp.float32)
    tab_ref[1] = dis * jnp.dot(genep_ref[...], w1,
                               preferred_element_type=jnp.float32)
    dis_ref[...] = dis


def _tc_a(gene_pad, gene_p, hist, w1):
    grid = (_NP // _BA,)
    return pl.pallas_call(
        _tca_body,
        grid=grid,
        in_specs=[
            pl.BlockSpec((_BA, _D0), lambda i: (i, 0)),
            pl.BlockSpec((_BA, _D0), lambda i: (i, 0)),
            pl.BlockSpec((2, _BA, _D0), lambda i: (0, i, 0)),
            pl.BlockSpec((_D0, _D1), lambda i: (0, 0)),
        ],
        out_specs=[
            pl.BlockSpec((2, _BA, _D1), lambda i: (0, i, 0)),
            pl.BlockSpec((_BA, 1), lambda i: (i, 0)),
        ],
        out_shape=[
            jax.ShapeDtypeStruct((2, _NP, _D1), jnp.float32),
            jax.ShapeDtypeStruct((_NP, 1), jnp.float32),
        ],
    )(gene_pad, gene_p, hist, w1)


def _tcb_body(agg_ref, dis_ref, b1_ref, w2_ref, tab_ref):
    dis = dis_ref[...]
    b1 = b1_ref[...]
    w2 = w2_ref[...]
    h0 = jnp.maximum(dis * agg_ref[0] + b1, 0.0)
    h1 = jnp.maximum(dis * agg_ref[1] + b1, 0.0)
    tab_ref[:, 0:_D2] = dis * jnp.dot(h0, w2,
                                      preferred_element_type=jnp.float32)
    tab_ref[:, _D2:2 * _D2] = dis * jnp.dot(h1, w2,
                                            preferred_element_type=jnp.float32)


def _tc_b(agg1, dis, b1, w2):
    grid = (_NP // _BA,)
    return pl.pallas_call(
        _tcb_body,
        grid=grid,
        in_specs=[
            pl.BlockSpec((2, _BA, _D1), lambda i: (0, i, 0)),
            pl.BlockSpec((_BA, 1), lambda i: (i, 0)),
            pl.BlockSpec((1, _D1), lambda i: (0, 0)),
            pl.BlockSpec((_D1, _D2), lambda i: (0, 0)),
        ],
        out_specs=pl.BlockSpec((_BA, 2 * _D2), lambda i: (i, 0)),
        out_shape=jax.ShapeDtypeStruct((_NP, 2 * _D2), jnp.float32),
    )(agg1, dis, b1, w2)


def _sc_agg2_body(src_hbm, dst_hbm, table_hbm, zeros_hbm, out_hbm,
                  sidx_v, didx_v, rows_v, sem, acc_sh):
    c = lax.axis_index("c")
    s = lax.axis_index("s")
    w = c * 16 + s

    @pl.when(c == 0)
    def _():
        pltpu.sync_copy(table_hbm.at[pl.ds(s * _RPT, _RPT)],
                        acc_sh.at[pl.ds(s * _RPT, _RPT)])

    @pl.when(c == 1)
    def _():
        pltpu.sync_copy(zeros_hbm.at[pl.ds(s * _RPT, _RPT)],
                        acc_sh.at[pl.ds(s * _RPT, _RPT)])

    plsc.subcore_barrier()

    def body(t, carry):
        base = w * _EPW + t * _CH
        pltpu.sync_copy(src_hbm.at[pl.ds(base, _CH)], sidx_v)
        pltpu.sync_copy(dst_hbm.at[pl.ds(base, _CH)], didx_v)
        pltpu.async_copy(table_hbm.at[sidx_v], rows_v, sem).wait()
        pltpu.sync_copy(rows_v, acc_sh.at[didx_v], add=True)
        return carry

    lax.fori_loop(0, _EPW // _CH, body, 0)
    plsc.subcore_barrier()
    pltpu.sync_copy(acc_sh.at[pl.ds(s * _RPT, _RPT)],
                    out_hbm.at[c, pl.ds(s * _RPT, _RPT)])


def _sc_agg2(src_pad, dst_pad, table, zeros_tab):
    mesh = plsc.VectorSubcoreMesh(core_axis_name="c", subcore_axis_name="s")
    f = pl.kernel(
        _sc_agg2_body,
        out_type=jax.ShapeDtypeStruct((2, _NP, 2 * _D2), jnp.float32),
        mesh=mesh,
        scratch_types=[
            pltpu.VMEM((_CH,), jnp.int32),
            pltpu.VMEM((_CH,), jnp.int32),
            pltpu.VMEM((_CH, 2 * _D2), jnp.float32),
            pltpu.SemaphoreType.DMA,
            pltpu.VMEM_SHARED((_NP, 2 * _D2), jnp.float32),
        ],
    )
    return f(src_pad, dst_pad, table, zeros_tab)


def _tcc_body(agg_ref, dis_ref, b2p_ref, c_ref):
    dis = dis_ref[...]
    b2p = b2p_ref[...]
    c_ref[...] = jnp.maximum(dis * (agg_ref[0] + agg_ref[1]) + b2p, 0.0)


def _tc_c(agg2, dis, b2p):
    grid = (_NP // _BA,)
    return pl.pallas_call(
        _tcc_body,
        grid=grid,
        in_specs=[
            pl.BlockSpec((2, _BA, 2 * _D2), lambda i: (0, i, 0)),
            pl.BlockSpec((_BA, 1), lambda i: (i, 0)),
            pl.BlockSpec((1, 2 * _D2), lambda i: (0, 0)),
        ],
        out_specs=pl.BlockSpec((_BA, 2 * _D2), lambda i: (i, 0)),
        out_shape=jax.ShapeDtypeStruct((_NP, 2 * _D2), jnp.float32),
    )(agg2, dis, b2p)


_BI = 400


def _tcd_body(mask_ref, cj_ref, ci_ref, wd_ref, bd_ref, ret1_ref, ret1c_ref):
    m = mask_ref[...]
    v = jnp.dot(m, cj_ref[...], preferred_element_type=jnp.float32)
    rs = jnp.sum(m, axis=1, keepdims=True)
    g = v / rs
    g1 = g[:, 0:_D2]
    g2 = g[:, _D2:2 * _D2]
    n1 = jnp.maximum(jnp.sqrt(jnp.sum(g1 * g1, axis=1, keepdims=True)), 1e-12)
    n2 = jnp.maximum(jnp.sqrt(jnp.sum(g2 * g2, axis=1, keepdims=True)), 1e-12)
    g1 = jax.nn.sigmoid(g1 / n1)
    g2 = jax.nn.sigmoid(g2 / n2)
    ci = ci_ref[...]
    wd = wd_ref[...]
    p1 = jnp.dot(ci[:, 0:_D2], wd, preferred_element_type=jnp.float32)
    p2 = jnp.dot(ci[:, _D2:2 * _D2], wd, preferred_element_type=jnp.float32)
    bd = bd_ref[0, 0]
    ret1_ref[:, 0:1] = jnp.sum(p1 * g1, axis=1, keepdims=True) + bd
    ret1_ref[:, 1:2] = jnp.sum(p2 * g1, axis=1, keepdims=True) + bd
    ret1c_ref[:, 0:1] = jnp.sum(p2 * g2, axis=1, keepdims=True) + bd
    ret1c_ref[:, 1:2] = jnp.sum(p1 * g2, axis=1, keepdims=True) + bd


def _tc_d(mask, c_nodes, wd0, bd):
    grid = (_N // _BI,)
    bd2 = bd.reshape(1, 1)
    return pl.pallas_call(
        _tcd_body,
        grid=grid,
        in_specs=[
            pl.BlockSpec((_BI, _N), lambda i: (i, 0)),
            pl.BlockSpec((_N, 2 * _D2), lambda i: (0, 0)),
            pl.BlockSpec((_BI, 2 * _D2), lambda i: (i, 0)),
            pl.BlockSpec((_D2, _D2), lambda i: (0, 0)),
            pl.BlockSpec((1, 1), lambda i: (0, 0)),
        ],
        out_specs=[
            pl.BlockSpec((_BI, 2), lambda i: (i, 0)),
            pl.BlockSpec((_BI, 2), lambda i: (i, 0)),
        ],
        out_shape=[
            jax.ShapeDtypeStruct((_N, 2), jnp.float32),
            jax.ShapeDtypeStruct((_N, 2), jnp.float32),
        ],
    )(mask, c_nodes, c_nodes, wd0, bd2)


def kernel(gene_data, spatial_edge_index, mask, W1, b1, W2, b2, Wd, bd):
    perm = jnp.asarray(_perm_np())

    src = spatial_edge_index[0]
    dst = spatial_edge_index[1]
    pad_e = _EP - _E
    src_pad = jnp.concatenate([src, jnp.zeros((pad_e,), jnp.int32)])
    dst_pad = jnp.concatenate([dst, jnp.full((pad_e,), _N, jnp.int32)])
    perm_pad = jnp.concatenate([perm, jnp.zeros((_NP - _N,), jnp.int32)])
    gene_pad = jnp.zeros((_NP, _D0), jnp.float32).at[:_N].set(gene_data)

    zeros_tab = jnp.zeros((_NP, _D0), jnp.float32)
    ones_tab = jnp.ones((_CH, _D0), jnp.float32)

    hist, gene_p = _sc_pass0(dst_pad, perm_pad, gene_pad, zeros_tab, ones_tab)

    tab1, dis = _tc_a(gene_pad, gene_p, hist, W1)
    agg1 = _sc_agg(src_pad, dst_pad, tab1.reshape(2 * _NP, _D1), _D1)
    tab2 = _tc_b(agg1, dis, b1.reshape(1, _D1), W2)
    agg2 = _sc_agg2(src_pad, dst_pad, tab2, zeros_tab)
    b2p = jnp.concatenate([b2, b2]).reshape(1, 2 * _D2)
    c_all = _tc_c(agg2, dis, b2p)

    c_nodes = c_all[:_N]
    ret1, ret1_c = _tc_d(mask, c_nodes, Wd[0], bd)

    x1 = c_nodes[:, 0:_D2]
    return (x1, ret1, ret1_c)
